# baseline (speedup 1.0000x reference)
import functools

import jax
import jax.numpy as jnp
from jax import lax
from jax.experimental import pallas as pl
from jax.experimental.pallas import tpu as pltpu
from jax.experimental.pallas import tpu_sc as plsc

N = 10000
E = 320000
D = 128

NC = 2
NS = 16
NW = NC * NS
C = 80
EP = 327680
EPW = EP // NW
NCHUNK = EPW // C
NPAD = 10240
RPT = NPAD // NS
NBUF = 4
GC = 16
NG = NCHUNK // GC

_sc_mesh = plsc.VectorSubcoreMesh(core_axis_name="c", subcore_axis_name="s")


@functools.partial(
    pl.kernel,
    out_type=jax.ShapeDtypeStruct((NC, NPAD, D), jnp.float32),
    mesh=_sc_mesh,
    scratch_types=[
        pltpu.VMEM_SHARED((NPAD, D), jnp.float32),
        pltpu.VMEM((GC, C), jnp.int32),
        pltpu.VMEM((GC, C), jnp.int32),
        pltpu.VMEM((C, D), jnp.float32),
        pltpu.VMEM((C, D), jnp.float32),
        pltpu.VMEM((C, D), jnp.float32),
        pltpu.VMEM((C, D), jnp.float32),
        pltpu.SemaphoreType.DMA,
        pltpu.SemaphoreType.DMA,
        pltpu.SemaphoreType.DMA,
        pltpu.SemaphoreType.DMA,
        pltpu.SemaphoreType.DMA,
        pltpu.SemaphoreType.DMA,
        pltpu.SemaphoreType.DMA,
        pltpu.SemaphoreType.DMA,
    ],
)
def _sc_aggregate(h_hbm, src_hbm, dst_hbm, out_hbm, agg, srcv, dstv, rows0,
                  rows1, rows2, rows3, gs0, gs1, gs2, gs3, ss0, ss1, ss2,
                  ss3):
    c = lax.axis_index("c")
    s = lax.axis_index("s")
    wid = s * NC + c
    rows = (rows0, rows1, rows2, rows3)
    gsem = (gs0, gs1, gs2, gs3)
    ssem = (ss0, ss1, ss2, ss3)

    zv = jnp.zeros((16,), jnp.float32)

    def _zero_row(i, carry):
        for j in range(D // 16):
            rows0[i, pl.ds(j * 16, 16)] = zv
        return carry

    lax.fori_loop(0, C, _zero_row, 0)
    for k in range(RPT // C):
        pltpu.sync_copy(rows0, agg.at[pl.ds(s * RPT + k * C, C)])

    plsc.subcore_barrier()

    def _gwait(b):
        pltpu.make_async_copy(h_hbm.at[srcv.at[0]], rows[b], gsem[b]).wait()

    def _swait(b):
        pltpu.make_async_copy(rows[b], agg.at[dstv.at[0]], ssem[b]).wait()

    def _group(g, carry):
        base = wid * NCHUNK + g * GC
        pltpu.sync_copy(src_hbm.at[pl.ds(base, GC)], srcv)
        pltpu.sync_copy(dst_hbm.at[pl.ds(base, GC)], dstv)
        pltpu.async_copy(h_hbm.at[srcv.at[0]], rows0, gs0)
        pltpu.async_copy(h_hbm.at[srcv.at[1]], rows1, gs1)

        def _round(r, carry2):
            for b in range(NBUF):
                k = r * NBUF + b
                j = k + 2
                bj = (b + 2) % NBUF

                @pl.when(j >= NBUF)
                def _():
                    _swait(bj)

                @pl.when(j < GC)
                def _():
                    pltpu.async_copy(h_hbm.at[srcv.at[j]], rows[bj],
                                     gsem[bj])

                _gwait(b)
                pltpu.async_copy(rows[b], agg.at[dstv.at[k]], ssem[b],
                                 add=True)
            return carry2

        lax.fori_loop(0, GC // NBUF, _round, 0)
        _swait(2)
        _swait(3)
        return carry

    lax.fori_loop(0, NG, _group, 0)

    plsc.subcore_barrier()
    pltpu.sync_copy(agg.at[pl.ds(s * RPT, RPT)],
                    out_hbm.at[c, pl.ds(s * RPT, RPT)])


BR = 1000


def _mlp_body(h_ref, p0_ref, p1_ref, w1_ref, b1_ref, w2_ref, b2_ref, o_ref):
    x = h_ref[...] + p0_ref[...] + p1_ref[...]
    t = jnp.dot(x, w1_ref[...], preferred_element_type=jnp.float32)
    t = jnp.maximum(t + b1_ref[...], 0.0)
    o = jnp.dot(t, w2_ref[...], preferred_element_type=jnp.float32)
    o_ref[...] = jnp.maximum(o + b2_ref[...], 0.0)


_mlp_call = pl.pallas_call(
    _mlp_body,
    grid=(N // BR,),
    in_specs=[
        pl.BlockSpec((BR, D), lambda i: (i, 0)),
        pl.BlockSpec((BR, D), lambda i: (i, 0)),
        pl.BlockSpec((BR, D), lambda i: (i, 0)),
        pl.BlockSpec((D, D), lambda i: (0, 0)),
        pl.BlockSpec((1, D), lambda i: (0, 0)),
        pl.BlockSpec((D, D), lambda i: (0, 0)),
        pl.BlockSpec((1, D), lambda i: (0, 0)),
    ],
    out_specs=pl.BlockSpec((BR, D), lambda i: (i, 0)),
    out_shape=jax.ShapeDtypeStruct((N, D), jnp.float32),
)


def kernel(features, edge_index, W1_0, b1_0, W2_0, b2_0, W1_1, b1_1, W2_1,
           b2_1):
    src = edge_index[0].astype(jnp.int32)
    dst = edge_index[1].astype(jnp.int32)
    pad = EP - E
    src = jnp.concatenate([src, jnp.zeros((pad,), jnp.int32)])
    dst = jnp.concatenate([dst, jnp.full((pad,), N, jnp.int32)])
    src = src.reshape(NW * NCHUNK, C)
    dst = dst.reshape(NW * NCHUNK, C)
    h = features
    for (W1, b1, W2, b2) in ((W1_0, b1_0, W2_0, b2_0),
                             (W1_1, b1_1, W2_1, b2_1)):
        parts = _sc_aggregate(h, src, dst)
        h = _mlp_call(h, parts[0, :N], parts[1, :N], W1, b1.reshape(1, D), W2,
                      b2.reshape(1, D))
    return h


# speedup vs baseline: 1.0635x; 1.0635x over previous
import functools

import jax
import jax.numpy as jnp
from jax import lax
from jax.experimental import pallas as pl
from jax.experimental.pallas import tpu as pltpu
from jax.experimental.pallas import tpu_sc as plsc

N = 10000
E = 320000
D = 128

NC = 2
NS = 16
NW = NC * NS
C = 128
EP = 327680
EPW = EP // NW
NCHUNK = EPW // C
NPAD = 10240
RPT = NPAD // NS
GC = 8
NG = NCHUNK // GC

_sc_mesh = plsc.VectorSubcoreMesh(core_axis_name="c", subcore_axis_name="s")


@functools.partial(
    pl.kernel,
    out_type=jax.ShapeDtypeStruct((NC, NPAD, D), jnp.float32),
    mesh=_sc_mesh,
    scratch_types=[
        pltpu.VMEM_SHARED((NPAD, D), jnp.float32),
        pltpu.VMEM((GC, C), jnp.int32),
        pltpu.VMEM((GC, C), jnp.int32),
        pltpu.VMEM((C, D), jnp.float32),
        pltpu.VMEM((C, D), jnp.float32),
        pltpu.SemaphoreType.DMA,
        pltpu.SemaphoreType.DMA,
        pltpu.SemaphoreType.DMA,
        pltpu.SemaphoreType.DMA,
    ],
)
def _sc_aggregate(h_hbm, src_hbm, dst_hbm, out_hbm, agg, srcv, dstv, rows0,
                  rows1, semg0, semg1, sems0, sems1):
    c = lax.axis_index("c")
    s = lax.axis_index("s")
    wid = s * NC + c

    zv = jnp.zeros((16,), jnp.float32)

    def _zero_row(i, carry):
        for j in range(D // 16):
            rows0[i, pl.ds(j * 16, 16)] = zv
        return carry

    lax.fori_loop(0, C, _zero_row, 0)
    for k in range(RPT // C):
        pltpu.sync_copy(rows0, agg.at[pl.ds(s * RPT + k * C, C)])

    plsc.subcore_barrier()

    bufs = (rows0, rows1)
    gsems = (semg0, semg1)
    ssems = (sems0, sems1)

    def _group(g, carry):
        base = wid * NCHUNK + g * GC
        pltpu.sync_copy(src_hbm.at[pl.ds(base, GC)], srcv)
        pltpu.sync_copy(dst_hbm.at[pl.ds(base, GC)], dstv)
        gd = {0: pltpu.async_copy(h_hbm.at[srcv.at[0]], rows0, semg0)}
        sd = {}
        for k in range(GC):
            p = k % 2
            gd[k].wait()
            sd[k] = pltpu.async_copy(bufs[p], agg.at[dstv.at[k]], ssems[p],
                                     add=True)
            if k + 1 < GC:
                if k >= 1:
                    sd[k - 1].wait()
                gd[k + 1] = pltpu.async_copy(h_hbm.at[srcv.at[k + 1]],
                                             bufs[1 - p], gsems[1 - p])
        sd[GC - 2].wait()
        sd[GC - 1].wait()
        return carry

    lax.fori_loop(0, NG, _group, 0)

    plsc.subcore_barrier()
    pltpu.sync_copy(agg.at[pl.ds(s * RPT, RPT)],
                    out_hbm.at[c, pl.ds(s * RPT, RPT)])


BR = 1000


def _mlp_body(h_ref, p0_ref, p1_ref, w1_ref, b1_ref, w2_ref, b2_ref, o_ref):
    x = h_ref[...] + p0_ref[...] + p1_ref[...]
    t = jnp.dot(x, w1_ref[...], preferred_element_type=jnp.float32)
    t = jnp.maximum(t + b1_ref[...], 0.0)
    o = jnp.dot(t, w2_ref[...], preferred_element_type=jnp.float32)
    o_ref[...] = jnp.maximum(o + b2_ref[...], 0.0)


_mlp_call = pl.pallas_call(
    _mlp_body,
    grid=(N // BR,),
    in_specs=[
        pl.BlockSpec((BR, D), lambda i: (i, 0)),
        pl.BlockSpec((BR, D), lambda i: (i, 0)),
        pl.BlockSpec((BR, D), lambda i: (i, 0)),
        pl.BlockSpec((D, D), lambda i: (0, 0)),
        pl.BlockSpec((1, D), lambda i: (0, 0)),
        pl.BlockSpec((D, D), lambda i: (0, 0)),
        pl.BlockSpec((1, D), lambda i: (0, 0)),
    ],
    out_specs=pl.BlockSpec((BR, D), lambda i: (i, 0)),
    out_shape=jax.ShapeDtypeStruct((N, D), jnp.float32),
)


def kernel(features, edge_index, W1_0, b1_0, W2_0, b2_0, W1_1, b1_1, W2_1,
           b2_1):
    src = edge_index[0].astype(jnp.int32)
    dst = edge_index[1].astype(jnp.int32)
    pad = EP - E
    src = jnp.concatenate([src, jnp.zeros((pad,), jnp.int32)])
    dst = jnp.concatenate([dst, jnp.full((pad,), N, jnp.int32)])
    src = src.reshape(NW * NCHUNK, C)
    dst = dst.reshape(NW * NCHUNK, C)
    h = features
    for (W1, b1, W2, b2) in ((W1_0, b1_0, W2_0, b2_0),
                             (W1_1, b1_1, W2_1, b2_1)):
        parts = _sc_aggregate(h, src, dst)
        h = _mlp_call(h, parts[0, :N], parts[1, :N], W1, b1.reshape(1, D), W2,
                      b2.reshape(1, D))
    return h


# speedup vs baseline: 1.3779x; 1.2956x over previous
import functools

import jax
import jax.numpy as jnp
from jax import lax
from jax.experimental import pallas as pl
from jax.experimental.pallas import tpu as pltpu
from jax.experimental.pallas import tpu_sc as plsc

N = 10000
E = 320000
D = 128

NC = 2
NS = 16
C = 32
EP = 327680
ECH = EP // C
CPS = ECH // NS
GC = 8
NG = CPS // GC
HALF = N // 2
NDUMP = 88
ACC_R = HALF + NDUMP
ZPT = 320

_sc_mesh = plsc.VectorSubcoreMesh(core_axis_name="c", subcore_axis_name="s")


@functools.partial(
    pl.kernel,
    out_type=jax.ShapeDtypeStruct((NC, ACC_R, D), jnp.float32),
    mesh=_sc_mesh,
    scratch_types=[
        pltpu.VMEM_SHARED((N, D), jnp.float32),
        pltpu.VMEM_SHARED((ACC_R, D), jnp.float32),
        pltpu.VMEM((GC, C), jnp.int32),
        pltpu.VMEM((GC, C), jnp.int32),
        pltpu.VMEM((C, D), jnp.float32),
        pltpu.VMEM((C, D), jnp.float32),
        pltpu.SemaphoreType.DMA,
        pltpu.SemaphoreType.DMA,
        pltpu.SemaphoreType.DMA,
        pltpu.SemaphoreType.DMA,
    ],
)
def _sc_aggregate(h_hbm, src_hbm, dst_hbm, out_hbm, h_sp, agg, srcv, dstv,
                  rows0, rows1, semg0, semg1, sems0, sems1):
    c = lax.axis_index("c")
    s = lax.axis_index("s")

    zv = jnp.zeros((16,), jnp.float32)

    def _zero_row(i, carry):
        for j in range(D // 16):
            rows0[i, pl.ds(j * 16, 16)] = zv
        return carry

    lax.fori_loop(0, C, _zero_row, 0)
    base0 = s * ZPT
    for k in range(9):
        pltpu.sync_copy(rows0, agg.at[pl.ds(base0 + k * C, C)])

    @pl.when(s < NS - 1)
    def _ztail():
        pltpu.sync_copy(rows0, agg.at[pl.ds(base0 + 9 * C, C)])

    for k in range(3):
        off = (s + 16 * k) * 200
        pltpu.sync_copy(h_hbm.at[pl.ds(off, 200)], h_sp.at[pl.ds(off, 200)])

    @pl.when(s < 2)
    def _tail():
        off = (48 + s) * 200
        pltpu.sync_copy(h_hbm.at[pl.ds(off, 200)], h_sp.at[pl.ds(off, 200)])

    plsc.subcore_barrier()

    bufs = (rows0, rows1)
    gsems = (semg0, semg1)
    ssems = (sems0, sems1)

    def _group(g, carry):
        base = s * CPS + g * GC
        pltpu.sync_copy(src_hbm.at[pl.ds(base, GC)], srcv)
        pltpu.sync_copy(dst_hbm.at[c, pl.ds(base, GC)], dstv)
        gd = {0: pltpu.async_copy(h_sp.at[srcv.at[0]], rows0, semg0)}
        sd = {}
        for k in range(GC):
            p = k % 2
            gd[k].wait()
            sd[k] = pltpu.async_copy(bufs[p], agg.at[dstv.at[k]], ssems[p],
                                     add=True)
            if k + 1 < GC:
                if k >= 1:
                    sd[k - 1].wait()
                gd[k + 1] = pltpu.async_copy(h_sp.at[srcv.at[k + 1]],
                                             bufs[1 - p], gsems[1 - p])
        sd[GC - 2].wait()
        sd[GC - 1].wait()
        return carry

    lax.fori_loop(0, NG, _group, 0)

    plsc.subcore_barrier()
    for k in range(9):
        pltpu.sync_copy(agg.at[pl.ds(base0 + k * C, C)],
                        out_hbm.at[c, pl.ds(base0 + k * C, C)])

    @pl.when(s < NS - 1)
    def _wtail():
        pltpu.sync_copy(agg.at[pl.ds(base0 + 9 * C, C)],
                        out_hbm.at[c, pl.ds(base0 + 9 * C, C)])


BR = 1000


def _mlp_body(h_ref, p_ref, w1_ref, b1_ref, w2_ref, b2_ref, o_ref):
    x = h_ref[...] + p_ref[...]
    t = jnp.dot(x, w1_ref[...], preferred_element_type=jnp.float32)
    t = jnp.maximum(t + b1_ref[...], 0.0)
    o = jnp.dot(t, w2_ref[...], preferred_element_type=jnp.float32)
    o_ref[...] = jnp.maximum(o + b2_ref[...], 0.0)


_mlp_call = pl.pallas_call(
    _mlp_body,
    grid=(N // BR,),
    in_specs=[
        pl.BlockSpec((BR, D), lambda i: (i, 0)),
        pl.BlockSpec((BR, D), lambda i: (i, 0)),
        pl.BlockSpec((D, D), lambda i: (0, 0)),
        pl.BlockSpec((1, D), lambda i: (0, 0)),
        pl.BlockSpec((D, D), lambda i: (0, 0)),
        pl.BlockSpec((1, D), lambda i: (0, 0)),
    ],
    out_specs=pl.BlockSpec((BR, D), lambda i: (i, 0)),
    out_shape=jax.ShapeDtypeStruct((N, D), jnp.float32),
)


def kernel(features, edge_index, W1_0, b1_0, W2_0, b2_0, W1_1, b1_1, W2_1,
           b2_1):
    src = edge_index[0].astype(jnp.int32)
    dst = edge_index[1].astype(jnp.int32)
    pad = EP - E
    src = jnp.concatenate([src, jnp.zeros((pad,), jnp.int32)])
    dst = jnp.concatenate([dst, jnp.full((pad,), -1, jnp.int32)])
    dump = HALF + (jnp.arange(EP, dtype=jnp.int32) % NDUMP)
    dst0 = jnp.where((dst >= 0) & (dst < HALF), dst, dump)
    dst1 = jnp.where(dst >= HALF, dst - HALF, dump)
    src = src.reshape(ECH, C)
    dsts = jnp.stack([dst0, dst1]).reshape(NC, ECH, C)
    h = features
    for (W1, b1, W2, b2) in ((W1_0, b1_0, W2_0, b2_0),
                             (W1_1, b1_1, W2_1, b2_1)):
        parts = _sc_aggregate(h, src, dsts)
        aggd = jnp.concatenate([parts[0, :HALF], parts[1, :HALF]], axis=0)
        h = _mlp_call(h, aggd, W1, b1.reshape(1, D), W2, b2.reshape(1, D))
    return h


# speedup vs baseline: 1.4410x; 1.0458x over previous
import functools

import jax
import jax.numpy as jnp
from jax import lax
from jax.experimental import pallas as pl
from jax.experimental.pallas import tpu as pltpu
from jax.experimental.pallas import tpu_sc as plsc

N = 10000
E = 320000
D = 128

NC = 2
NS = 16
C = 32
EP = 327680
ECH = EP // C
CPS = ECH // NS
GC = 8
NG = CPS // GC
HALF = N // 2
NDUMP = 88
ACC_R = HALF + NDUMP
ZPT = 320

_sc_mesh = plsc.VectorSubcoreMesh(core_axis_name="c", subcore_axis_name="s")


@functools.partial(
    pl.kernel,
    out_type=jax.ShapeDtypeStruct((NC, ACC_R, D), jnp.float32),
    mesh=_sc_mesh,
    scratch_types=[
        pltpu.VMEM_SHARED((N, D), jnp.float32),
        pltpu.VMEM_SHARED((ACC_R, D), jnp.float32),
        pltpu.VMEM((GC, 2, C), jnp.int32),
        pltpu.VMEM((C, D), jnp.float32),
        pltpu.VMEM((C, D), jnp.float32),
        pltpu.SemaphoreType.DMA,
        pltpu.SemaphoreType.DMA,
        pltpu.SemaphoreType.DMA,
        pltpu.SemaphoreType.DMA,
    ],
)
def _sc_aggregate(h_hbm, idx_hbm, out_hbm, h_sp, agg, idxv,
                  rows0, rows1, semg0, semg1, sems0, sems1):
    c = lax.axis_index("c")
    s = lax.axis_index("s")

    zv = jnp.zeros((16,), jnp.float32)

    def _zero_row(i, carry):
        for j in range(D // 16):
            rows0[i, pl.ds(j * 16, 16)] = zv
        return carry

    lax.fori_loop(0, C, _zero_row, 0)
    base0 = s * ZPT
    for k in range(9):
        pltpu.sync_copy(rows0, agg.at[pl.ds(base0 + k * C, C)])

    @pl.when(s < NS - 1)
    def _ztail():
        pltpu.sync_copy(rows0, agg.at[pl.ds(base0 + 9 * C, C)])

    for k in range(3):
        off = (s + 16 * k) * 200
        pltpu.sync_copy(h_hbm.at[pl.ds(off, 200)], h_sp.at[pl.ds(off, 200)])

    @pl.when(s < 2)
    def _tail():
        off = (48 + s) * 200
        pltpu.sync_copy(h_hbm.at[pl.ds(off, 200)], h_sp.at[pl.ds(off, 200)])

    plsc.subcore_barrier()

    bufs = (rows0, rows1)
    gsems = (semg0, semg1)
    ssems = (sems0, sems1)

    def _group(g, carry):
        base = s * CPS + g * GC
        pltpu.sync_copy(idx_hbm.at[c, pl.ds(base, GC)], idxv)
        gd = {0: pltpu.async_copy(h_sp.at[idxv.at[0, 0]], rows0, semg0)}
        sd = {}
        for k in range(GC):
            p = k % 2
            gd[k].wait()
            sd[k] = pltpu.async_copy(bufs[p], agg.at[idxv.at[k, 1]], ssems[p],
                                     add=True)
            if k + 1 < GC:
                if k >= 1:
                    sd[k - 1].wait()
                gd[k + 1] = pltpu.async_copy(h_sp.at[idxv.at[k + 1, 0]],
                                             bufs[1 - p], gsems[1 - p])
        sd[GC - 2].wait()
        sd[GC - 1].wait()
        return carry

    lax.fori_loop(0, NG, _group, 0)

    plsc.subcore_barrier()
    for k in range(9):
        pltpu.sync_copy(agg.at[pl.ds(base0 + k * C, C)],
                        out_hbm.at[c, pl.ds(base0 + k * C, C)])

    @pl.when(s < NS - 1)
    def _wtail():
        pltpu.sync_copy(agg.at[pl.ds(base0 + 9 * C, C)],
                        out_hbm.at[c, pl.ds(base0 + 9 * C, C)])


BR = 1000


def _mlp_body(h_ref, p_ref, w1_ref, b1_ref, w2_ref, b2_ref, o_ref):
    x = h_ref[...] + p_ref[...]
    t = jnp.dot(x, w1_ref[...], preferred_element_type=jnp.float32)
    t = jnp.maximum(t + b1_ref[...], 0.0)
    o = jnp.dot(t, w2_ref[...], preferred_element_type=jnp.float32)
    o_ref[...] = jnp.maximum(o + b2_ref[...], 0.0)


_mlp_call = pl.pallas_call(
    _mlp_body,
    grid=(N // BR,),
    in_specs=[
        pl.BlockSpec((BR, D), lambda i: (i, 0)),
        pl.BlockSpec((BR, D), lambda i: (i, 0)),
        pl.BlockSpec((D, D), lambda i: (0, 0)),
        pl.BlockSpec((1, D), lambda i: (0, 0)),
        pl.BlockSpec((D, D), lambda i: (0, 0)),
        pl.BlockSpec((1, D), lambda i: (0, 0)),
    ],
    out_specs=pl.BlockSpec((BR, D), lambda i: (i, 0)),
    out_shape=jax.ShapeDtypeStruct((N, D), jnp.float32),
)


def kernel(features, edge_index, W1_0, b1_0, W2_0, b2_0, W1_1, b1_1, W2_1,
           b2_1):
    src = edge_index[0].astype(jnp.int32)
    dst = edge_index[1].astype(jnp.int32)
    pad = EP - E
    src = jnp.concatenate([src, jnp.zeros((pad,), jnp.int32)])
    dst = jnp.concatenate([dst, jnp.full((pad,), -1, jnp.int32)])
    dump = HALF + (jnp.arange(EP, dtype=jnp.int32) % NDUMP)
    dst0 = jnp.where((dst >= 0) & (dst < HALF), dst, dump)
    dst1 = jnp.where(dst >= HALF, dst - HALF, dump)
    srcc = src.reshape(ECH, 1, C)
    idx = jnp.stack([
        jnp.concatenate([srcc, dst0.reshape(ECH, 1, C)], axis=1),
        jnp.concatenate([srcc, dst1.reshape(ECH, 1, C)], axis=1),
    ])
    h = features
    for (W1, b1, W2, b2) in ((W1_0, b1_0, W2_0, b2_0),
                             (W1_1, b1_1, W2_1, b2_1)):
        parts = _sc_aggregate(h, idx)
        aggd = jnp.concatenate([parts[0, :HALF], parts[1, :HALF]], axis=0)
        h = _mlp_call(h, aggd, W1, b1.reshape(1, D), W2, b2.reshape(1, D))
    return h
